# baseline (speedup 1.0000x reference)
import jax
import jax.numpy as jnp
from jax.experimental import pallas as pl
from jax.experimental.pallas import tpu as pltpu

_BB = 8


def _body(months_ref, ch_ref, pos_ref, mtab_ref, tok_ref, out_ref):
    tok = tok_ref[...]
    bb, t = tok.shape[0], tok.shape[1]
    m = months_ref[0]
    mo = jnp.zeros((bb, t, 2, 128), jnp.float32)
    for k in range(12):
        sel = (m == k).astype(jnp.float32)[:, :, None, None]
        mo = mo + sel * mtab_ref[k][None, None, :, :]
    ch = ch_ref[...]
    pos = pos_ref[...]
    out_ref[:, :, :, 0:2] = tok[:, :, :, 0:2] + ch[None, None, :, :, :]
    out_ref[:, :, :, 2:4] = tok[:, :, :, 2:4] + pos[None, :, None, :, :]
    out_ref[:, :, :, 4:6] = tok[:, :, :, 4:6] + mo[:, :, None, :, :]
    out_ref[:, :, :, 6:8] = tok[:, :, :, 6:8]


@jax.jit
def kernel(modality_tokens, timestamps, channel_embed, pos_embed, month_tab):
    b, t, bs, d = modality_tokens.shape
    months = timestamps[:, :, 1].astype(jnp.int32).reshape(b // _BB, _BB, t)
    tok5 = modality_tokens.reshape(b, t, bs, 8, 128)
    out = pl.pallas_call(
        _body,
        grid=(b // _BB,),
        in_specs=[
            pl.BlockSpec((1, _BB, t), lambda i: (i, 0, 0)),
            pl.BlockSpec((bs, 2, 128), lambda i: (0, 0, 0)),
            pl.BlockSpec((t, 2, 128), lambda i: (0, 0, 0)),
            pl.BlockSpec((12, 2, 128), lambda i: (0, 0, 0)),
            pl.BlockSpec((_BB, t, bs, 8, 128), lambda i: (i, 0, 0, 0, 0)),
        ],
        out_specs=pl.BlockSpec((_BB, t, bs, 8, 128), lambda i: (i, 0, 0, 0, 0)),
        out_shape=jax.ShapeDtypeStruct((b, t, bs, 8, 128), jnp.float32),
        compiler_params=pltpu.CompilerParams(
            dimension_semantics=("arbitrary",),
            vmem_limit_bytes=100 * 1024 * 1024,
        ),
    )(months, channel_embed.reshape(bs, 2, 128), pos_embed[:t].reshape(t, 2, 128),
      month_tab.reshape(12, 2, 128), tok5)
    return out.reshape(b, t, bs, d)
